# Initial kernel scaffold; baseline (speedup 1.0000x reference)
import jax
import jax.numpy as jnp
from jax.experimental import pallas as pl


def kernel(x, src_idx, dst_idx, path_len, b):
    raise NotImplementedError("write your pallas kernel here")


# speedup vs baseline: 4.5484x; 4.5484x over previous
import functools

import jax
import jax.numpy as jnp
from jax import lax
from jax.experimental import pallas as pl
from jax.experimental.pallas import tpu as pltpu
from jax.experimental.pallas import tpu_sc as plsc

N = 4096
P = 262144
MPD = 20
L = 16
NC = 2
NS = 16
CH = P // NS
SB = 4096
NG = 256
GPT = 8
GR = 16
EB = 2048
PAD = P + EB


def _body(skey_hbm, sval_hbm, out_hbm,
          keyb, cnt, cntall, startb, slab, ekey, eval_,
          cntS, sem):
  core = lax.axis_index("c")
  sub = lax.axis_index("s")
  lanes = lax.iota(jnp.int32, L)
  zeros16i = jnp.zeros((L,), jnp.int32)
  ones16i = jnp.ones((L,), jnp.int32)
  zeros16f = jnp.zeros((L,), jnp.float32)

  for g in range(NG // L):
    cnt[pl.ds(g * L, L)] = zeros16i

  def hist_sb(bb, _):
    off = sub * CH + bb * SB
    pltpu.sync_copy(skey_hbm.at[pl.ds(off, SB)], keyb)

    def hist_v(j, _):
      k16 = keyb[pl.ds(j * L, L)]
      bucket = lax.shift_right_logical(k16, 16)
      plsc.addupdate_scatter(cnt, [bucket], ones16i)
      return 0

    lax.fori_loop(0, SB // L, hist_v, 0)
    return 0

  lax.fori_loop(0, CH // SB, hist_sb, 0)

  pltpu.sync_copy(cnt, cntS.at[pl.ds(sub * NG, NG)])
  plsc.subcore_barrier()
  pltpu.sync_copy(cntS, cntall)

  run = jnp.int32(0)
  for g in range(NG // L):
    tot_g = zeros16i
    for t in range(NS):
      tot_g = tot_g + cntall[pl.ds(t * NG + g * L, L)]
    ex = plsc.cumsum(tot_g) - tot_g
    startb[pl.ds(g * L, L)] = ex + run
    run = run + jnp.sum(tot_g)
  startb[pl.ds(NG, L)] = jnp.zeros((L,), jnp.int32) + run

  def zero_v(t, _):
    slab[t >> 8, pl.ds((t & 255) * L, L)] = zeros16f
    return 0

  lax.fori_loop(0, GR * (N // L), zero_v, 0)

  tile_id = core * NS + sub

  def extract(vec_ref, i):
    gbase = pl.multiple_of((i >> 4) << 4, 16)
    v = vec_ref[pl.ds(gbase, L)]
    return jnp.sum(jnp.where(lanes == (i & 15), v, zeros16i))

  def group_body(g, _):
    gb = tile_id * GPT + g
    start = extract(startb, gb)
    end = extract(startb, gb + 1)
    base0 = pl.multiple_of((start >> 4) << 4, 16)
    nb = (end - base0 + EB - 1) // EB

    def apply_batch(k, _):
      boff = pl.multiple_of(base0 + k * EB, 16)
      pltpu.sync_copy(skey_hbm.at[pl.ds(boff, EB)], ekey)
      pltpu.sync_copy(sval_hbm.at[pl.ds(boff, EB)], eval_)

      def apply_v(v, _):
        gi = boff + v * L + lanes
        live = (gi >= start) & (gi < end)
        k16 = ekey[pl.ds(v * L, L)]
        r16 = lax.shift_right_logical(k16, 12) & (GR - 1)
        c16 = k16 & (N - 1)
        _c, last = plsc.scan_count(k16, mask=live)
        val = eval_[pl.ds(v * L, L)]
        plsc.store_scatter(slab, [r16, c16], val, mask=last & live)
        return 0

      lax.fori_loop(0, EB // L, apply_v, 0)
      return 0

    lax.fori_loop(0, nb, apply_batch, 0)

    row0 = tile_id * (GPT * GR) + g * GR
    pltpu.async_copy(
        slab, out_hbm.at[pl.ds(row0, GR), pl.ds(0, N)], sem).wait()

    def restore_batch(k, _):
      boff = pl.multiple_of(base0 + k * EB, 16)
      pltpu.sync_copy(skey_hbm.at[pl.ds(boff, EB)], ekey)

      def rest_v(v, _):
        gi = boff + v * L + lanes
        live = (gi >= start) & (gi < end)
        k16 = ekey[pl.ds(v * L, L)]
        r16 = lax.shift_right_logical(k16, 12) & (GR - 1)
        c16 = k16 & (N - 1)
        plsc.store_scatter(slab, [r16, c16], zeros16f, mask=live)
        return 0

      lax.fori_loop(0, EB // L, rest_v, 0)
      return 0

    lax.fori_loop(0, nb, restore_batch, 0)
    return 0

  lax.fori_loop(0, GPT, group_body, 0)


@jax.jit
def _sc_scatter(skey, sval):
  mesh = plsc.VectorSubcoreMesh(core_axis_name="c", subcore_axis_name="s")
  return pl.kernel(
      _body,
      out_type=jax.ShapeDtypeStruct((N, N), jnp.float32),
      mesh=mesh,
      compiler_params=pltpu.CompilerParams(needs_layout_passes=False),
      scratch_types=[
          pltpu.VMEM((SB,), jnp.int32),
          pltpu.VMEM((NG,), jnp.int32),
          pltpu.VMEM((NS * NG,), jnp.int32),
          pltpu.VMEM((NG + L,), jnp.int32),
          pltpu.VMEM((GR, N), jnp.float32),
          pltpu.VMEM((EB,), jnp.int32),
          pltpu.VMEM((EB,), jnp.float32),
          pltpu.VMEM_SHARED((NS * NG,), jnp.int32),
          pltpu.SemaphoreType.DMA,
      ],
  )(skey, sval)


def kernel(x, src_idx, dst_idx, path_len, b):
  del x
  src32 = src_idx.astype(jnp.int32)
  dst32 = dst_idx.astype(jnp.int32)
  key = src32 * N + dst32
  idx = jnp.clip(jnp.minimum(path_len, MPD) - 1, 0, MPD - 1)
  vals = jnp.take(b, idx, axis=0)
  skey, sval = lax.sort((key, vals), dimension=0, is_stable=False, num_keys=1)
  skey = jnp.pad(skey, (0, PAD - P), constant_values=jnp.int32(2**24))
  sval = jnp.pad(sval, (0, PAD - P))
  return _sc_scatter(skey, sval)


# speedup vs baseline: 4.5506x; 1.0005x over previous
import jax
import jax.numpy as jnp
from jax import lax
from jax.experimental import pallas as pl
from jax.experimental.pallas import tpu as pltpu
from jax.experimental.pallas import tpu_sc as plsc

N = 4096
P = 262144
MPD = 20
L = 16
NC = 2
NS = 16
CH = P // NS
SB = 4096
NG = 256
GPT = 8
GR = 16
EB = 2048
PAD = P + EB


def _body(skey_hbm, sval_hbm, out_hbm,
          keyb, cnt, cntall, startb, slab, ekey, eval_,
          cntS, sem):
  core = lax.axis_index("c")
  sub = lax.axis_index("s")
  lanes = lax.iota(jnp.int32, L)
  zeros16i = jnp.zeros((L,), jnp.int32)
  ones16i = jnp.ones((L,), jnp.int32)
  zeros16f = jnp.zeros((L,), jnp.float32)

  for g in range(NG // L):
    cnt[pl.ds(g * L, L)] = zeros16i

  def hist_sb(bb, _):
    off = sub * CH + bb * SB
    pltpu.sync_copy(skey_hbm.at[pl.ds(off, SB)], keyb)

    def hist_v(j, _):
      k16 = keyb[pl.ds(j * L, L)]
      bucket = lax.shift_right_logical(k16, 16)
      plsc.addupdate_scatter(cnt, [bucket], ones16i)
      return 0

    lax.fori_loop(0, SB // L, hist_v, 0)
    return 0

  lax.fori_loop(0, CH // SB, hist_sb, 0)

  pltpu.sync_copy(cnt, cntS.at[pl.ds(sub * NG, NG)])
  plsc.subcore_barrier()
  pltpu.sync_copy(cntS, cntall)

  run = jnp.int32(0)
  for g in range(NG // L):
    tot_g = zeros16i
    for t in range(NS):
      tot_g = tot_g + cntall[pl.ds(t * NG + g * L, L)]
    ex = plsc.cumsum(tot_g) - tot_g
    startb[pl.ds(g * L, L)] = ex + run
    run = run + jnp.sum(tot_g)
  startb[pl.ds(NG, L)] = jnp.zeros((L,), jnp.int32) + run

  def zero_v(t, _):
    slab[t >> 8, pl.ds((t & 255) * L, L)] = zeros16f
    return 0

  lax.fori_loop(0, GR * (N // L), zero_v, 0)

  tile_id = core * NS + sub

  def extract(vec_ref, i):
    gbase = pl.multiple_of((i >> 4) << 4, 16)
    v = vec_ref[pl.ds(gbase, L)]
    return jnp.sum(jnp.where(lanes == (i & 15), v, zeros16i))

  def group_body(g, _):
    gb = tile_id * GPT + g
    start = extract(startb, gb)
    end = extract(startb, gb + 1)
    base0 = pl.multiple_of((start >> 4) << 4, 16)
    nb = (end - base0 + EB - 1) // EB

    def apply_batch(k, _):
      boff = pl.multiple_of(base0 + k * EB, 16)
      pltpu.sync_copy(skey_hbm.at[pl.ds(boff, EB)], ekey)
      pltpu.sync_copy(sval_hbm.at[pl.ds(boff, EB)], eval_)

      def apply_v(v, _):
        gi = boff + v * L + lanes
        live = (gi >= start) & (gi < end)
        k16 = ekey[pl.ds(v * L, L)]
        r16 = lax.shift_right_logical(k16, 12) & (GR - 1)
        c16 = k16 & (N - 1)
        _c, last = plsc.scan_count(k16, mask=live)
        val = eval_[pl.ds(v * L, L)]
        plsc.store_scatter(slab, [r16, c16], val, mask=last & live)
        return 0

      lax.fori_loop(0, EB // L, apply_v, 0)
      return 0

    lax.fori_loop(0, nb, apply_batch, 0)

    row0 = tile_id * (GPT * GR) + g * GR
    pltpu.async_copy(
        slab, out_hbm.at[pl.ds(row0, GR), pl.ds(0, N)], sem).wait()

    def restore_batch(k, _):
      boff = pl.multiple_of(base0 + k * EB, 16)
      pltpu.sync_copy(skey_hbm.at[pl.ds(boff, EB)], ekey)

      def rest_v(v, _):
        gi = boff + v * L + lanes
        live = (gi >= start) & (gi < end)
        k16 = ekey[pl.ds(v * L, L)]
        r16 = lax.shift_right_logical(k16, 12) & (GR - 1)
        c16 = k16 & (N - 1)
        plsc.store_scatter(slab, [r16, c16], zeros16f, mask=live)
        return 0

      lax.fori_loop(0, EB // L, rest_v, 0)
      return 0

    lax.fori_loop(0, nb, restore_batch, 0)
    return 0

  lax.fori_loop(0, GPT, group_body, 0)


@jax.jit
def _sc_scatter(skey, sval):
  mesh = plsc.VectorSubcoreMesh(core_axis_name="c", subcore_axis_name="s")
  return pl.kernel(
      _body,
      out_type=jax.ShapeDtypeStruct((N, N), jnp.float32),
      mesh=mesh,
      compiler_params=pltpu.CompilerParams(needs_layout_passes=False),
      scratch_types=[
          pltpu.VMEM((SB,), jnp.int32),
          pltpu.VMEM((NG,), jnp.int32),
          pltpu.VMEM((NS * NG,), jnp.int32),
          pltpu.VMEM((NG + L,), jnp.int32),
          pltpu.VMEM((GR, N), jnp.float32),
          pltpu.VMEM((EB,), jnp.int32),
          pltpu.VMEM((EB,), jnp.float32),
          pltpu.VMEM_SHARED((NS * NG,), jnp.int32),
          pltpu.SemaphoreType.DMA,
      ],
  )(skey, sval)


def kernel(x, src_idx, dst_idx, path_len, b):
  del x
  src32 = src_idx.astype(jnp.int32)
  dst32 = dst_idx.astype(jnp.int32)
  key = src32 * N + dst32
  idx = jnp.clip(jnp.minimum(path_len, MPD) - 1, 0, MPD - 1)
  vals = jnp.take(b, idx, axis=0)
  skey, sval = lax.sort((key, vals), dimension=0, is_stable=False, num_keys=1)
  skey = jnp.pad(skey, (0, PAD - P), constant_values=jnp.int32(2**24))
  sval = jnp.pad(sval, (0, PAD - P))
  return _sc_scatter(skey, sval)
